# baseline (speedup 1.0000x reference)
import functools

import jax
import jax.numpy as jnp
from jax import lax
from jax.experimental import pallas as pl
from jax.experimental.pallas import tpu as pltpu
from jax.experimental.pallas import tpu_sc as plsc

N_NODES = 50000
N_EDGES = 800000
N_GRAPHS = 64
BN = 512
NPAD = 50176
NBLK = NPAD // BN

NC = 2
NS = 16
ET = N_EDGES // NS
CH = 1024
K = 64
SEL = CH + K
ZR = 16


def _perm32(x):
    bn, n = x.shape
    return x.reshape(bn, n // 32, 16, 2).transpose(0, 1, 3, 2).reshape(bn, n)


def _iperm32(x):
    bn, n = x.shape
    return x.reshape(bn, n // 32, 2, 16).transpose(0, 1, 3, 2).reshape(bn, n)


def _perm_cols(t, n):
    return jnp.concatenate([_perm32(t[:, :n]), t[:, n:]], axis=1)


def _stage_a_body(x_ref, w1_ref, asr_ref, ads_ref, tsrc_ref, tdst_ref, self_ref):
    h = jnp.dot(x_ref[...], w1_ref[...], preferred_element_type=jnp.float32)
    hr = h.reshape(BN, 4, 64)
    a_src = jnp.sum(hr * asr_ref[...][None, :, :], axis=-1)
    a_dst = jnp.sum(hr * ads_ref[...][None, :, :], axis=-1)
    abits = jnp.stack([a_src.astype(jnp.bfloat16),
                       jnp.zeros((BN, 4), jnp.bfloat16)], axis=-1).reshape(BN, 8)
    tsrc_ref[...] = jnp.concatenate(
        [h.astype(jnp.bfloat16), abits,
         jnp.zeros((BN, 24), jnp.bfloat16)], axis=1)
    tdst_ref[...] = a_dst
    al = a_src + a_dst
    w = jnp.exp(jnp.maximum(al, 0.2 * al))
    wb = jnp.broadcast_to(w[:, :, None], (BN, 4, 64)).reshape(BN, 256)
    self_ref[...] = jnp.concatenate(
        [h * wb, w, jnp.zeros((BN, 12), jnp.float32)], axis=1)


def _stage_a(x, W1, att_src1, att_dst1):
    xp = jnp.pad(x, ((0, NPAD - N_NODES), (0, 0)))
    return pl.pallas_call(
        _stage_a_body,
        grid=(NBLK,),
        in_specs=[
            pl.BlockSpec((BN, 58), lambda i: (i, 0)),
            pl.BlockSpec((58, 256), lambda i: (0, 0)),
            pl.BlockSpec((4, 64), lambda i: (0, 0)),
            pl.BlockSpec((4, 64), lambda i: (0, 0)),
        ],
        out_specs=[
            pl.BlockSpec((BN, 288), lambda i: (i, 0)),
            pl.BlockSpec((BN, 4), lambda i: (i, 0)),
            pl.BlockSpec((BN, 272), lambda i: (i, 0)),
        ],
        out_shape=[
            jax.ShapeDtypeStruct((NPAD, 288), jnp.bfloat16),
            jax.ShapeDtypeStruct((NPAD, 4), jnp.float32),
            jax.ShapeDtypeStruct((NPAD, 272), jnp.float32),
        ],
    )(xp, W1, att_src1, att_dst1)


def _stage_c_body(acc_ref, self_ref, b1_ref, w2_ref, asr_ref, ads_ref,
                  tsrc_ref, tdst_ref, self2_ref):
    tot = acc_ref[...] + self_ref[...]
    num = tot[:, :256]
    den = tot[:, 256:260]
    denb = jnp.broadcast_to(den[:, :, None], (BN, 4, 64)).reshape(BN, 256)
    h1 = jnp.maximum(jnp.where(denb > 0, num / denb, 0.0) + b1_ref[...], 0.0)
    h2 = jnp.dot(h1, w2_ref[...], preferred_element_type=jnp.float32)
    a_src = jnp.sum(h2 * asr_ref[...], axis=-1, keepdims=True)
    a_dst = jnp.sum(h2 * ads_ref[...], axis=-1, keepdims=True)
    abits = jnp.concatenate([a_src.astype(jnp.bfloat16),
                             jnp.zeros((BN, 1), jnp.bfloat16)], axis=1)
    tsrc_ref[...] = jnp.concatenate(
        [h2.astype(jnp.bfloat16), abits,
         jnp.zeros((BN, 30), jnp.bfloat16)], axis=1)
    tdst_ref[...] = jnp.concatenate(
        [a_dst, jnp.zeros((BN, 3), jnp.float32)], axis=1)
    al = a_src + a_dst
    w = jnp.exp(jnp.maximum(al, 0.2 * al))
    self2_ref[...] = jnp.concatenate(
        [h2 * w, w, jnp.zeros((BN, 15), jnp.float32)], axis=1)


def _stage_c(acc1, self1, b1p, W2p, att_src2, att_dst2):
    return pl.pallas_call(
        _stage_c_body,
        grid=(NBLK,),
        in_specs=[
            pl.BlockSpec((BN, 272), lambda i: (i, 0)),
            pl.BlockSpec((BN, 272), lambda i: (i, 0)),
            pl.BlockSpec((1, 256), lambda i: (0, 0)),
            pl.BlockSpec((256, 128), lambda i: (0, 0)),
            pl.BlockSpec((1, 128), lambda i: (0, 0)),
            pl.BlockSpec((1, 128), lambda i: (0, 0)),
        ],
        out_specs=[
            pl.BlockSpec((BN, 160), lambda i: (i, 0)),
            pl.BlockSpec((BN, 4), lambda i: (i, 0)),
            pl.BlockSpec((BN, 144), lambda i: (i, 0)),
        ],
        out_shape=[
            jax.ShapeDtypeStruct((NPAD, 160), jnp.bfloat16),
            jax.ShapeDtypeStruct((NPAD, 4), jnp.float32),
            jax.ShapeDtypeStruct((NPAD, 144), jnp.float32),
        ],
    )(acc1, self1, b1p[None], W2p, att_src2, att_dst2)


def _stage_e_body(acc_ref, self_ref, b2_ref, batch_ref, g_ref, be_ref,
                  out_ref, pooled_ref):
    i = pl.program_id(0)

    @pl.when(i == 0)
    def _():
        pooled_ref[...] = jnp.zeros((N_GRAPHS, 256), jnp.float32)

    tot = acc_ref[...] + self_ref[...]
    den = tot[:, 128:129]
    h2 = jnp.maximum(jnp.where(den > 0, tot[:, :128] / den, 0.0) + b2_ref[...],
                     0.0)
    bvec = batch_ref[0, 0, :]
    gids = lax.broadcasted_iota(jnp.int32, (N_GRAPHS, BN), 0)
    m = (bvec[None, :] == gids).astype(jnp.float32)
    hext = jnp.concatenate(
        [h2, jnp.ones((BN, 1), jnp.float32), jnp.zeros((BN, 127), jnp.float32)],
        axis=1)
    pooled_ref[...] += jnp.dot(m, hext, preferred_element_type=jnp.float32)

    @pl.when(i == NBLK - 1)
    def _():
        acc = pooled_ref[...]
        sums = acc[:, :128]
        counts = acc[:, 128:129]
        pooled = sums / jnp.clip(counts, 1.0, None)
        mu = jnp.mean(pooled, axis=0, keepdims=True)
        var = jnp.mean((pooled - mu) ** 2, axis=0, keepdims=True)
        norm = (pooled - mu) / jnp.sqrt(var + 1e-5)
        out_ref[...] = norm * g_ref[...] + be_ref[...]


def _stage_e(acc2, self2, b2p, batch, gamma, beta):
    bpad = jnp.pad(batch, (0, NPAD - N_NODES), constant_values=N_GRAPHS)
    b3d = bpad.reshape(NBLK, BN)[:, None, :]
    return pl.pallas_call(
        _stage_e_body,
        grid=(NBLK,),
        in_specs=[
            pl.BlockSpec((BN, 144), lambda i: (i, 0)),
            pl.BlockSpec((BN, 144), lambda i: (i, 0)),
            pl.BlockSpec((1, 128), lambda i: (0, 0)),
            pl.BlockSpec((1, 1, BN), lambda i: (i, 0, 0)),
            pl.BlockSpec((1, 128), lambda i: (0, 0)),
            pl.BlockSpec((1, 128), lambda i: (0, 0)),
        ],
        out_specs=pl.BlockSpec((N_GRAPHS, 128), lambda i: (0, 0)),
        out_shape=jax.ShapeDtypeStruct((N_GRAPHS, 128), jnp.float32),
        scratch_shapes=[pltpu.VMEM((N_GRAPHS, 256), jnp.float32)],
    )(acc2, self2, b2p[None], b3d, gamma[None], beta[None])


def _make_sc_edge_kernel(F, FB, NH, R, PASSES):
    FA = F - 16
    FH = FA // NH
    TR = R // NS
    NPR = PASSES * 2 * R

    mesh = plsc.VectorSubcoreMesh(core_axis_name="c", subcore_axis_name="s")

    @functools.partial(
        pl.kernel,
        out_type=jax.ShapeDtypeStruct((NPR, F), jnp.float32),
        mesh=mesh,
        compiler_params=pltpu.CompilerParams(needs_layout_passes=False,
                                             use_tc_tiling_on_sc=False),
        scratch_types=[
            pltpu.VMEM((CH,), jnp.int32),
            pltpu.VMEM((CH,), jnp.int32),
            pltpu.VMEM((SEL,), jnp.int32),
            pltpu.VMEM((1, K), jnp.int32),
            pltpu.VMEM((1, K), jnp.int32),
            pltpu.VMEM((K, FB), jnp.bfloat16),
            pltpu.VMEM(((R + 8) * 4,), jnp.float32),
            pltpu.VMEM((K, F), jnp.float32),
            pltpu.VMEM((ZR, F), jnp.float32),
            pltpu.VMEM_SHARED((R + 8, F), jnp.float32),
            pltpu.SemaphoreType.DMA,
            pltpu.SemaphoreType.DMA,
        ],
    )
    def sc_kernel(src_hbm, dst_hbm, tsrc_hbm, tdst_hbm, out_hbm,
                  src_v, dst_v, selp, gidx, sidx, rows, adst_v, msg,
                  zeros_v, acc, sem1, sem2):
        cid = lax.axis_index("c")
        sid = lax.axis_index("s")
        lanes = lax.iota(jnp.int32, 16)
        zf = jnp.zeros((16,), jnp.float32)

        def zrow(i, _):
            for j in range(F // 16):
                zeros_v[i, pl.ds(j * 16, 16)] = zf
            return 0
        lax.fori_loop(0, ZR, zrow, 0)

        edge_base = sid * ET

        def process_chunk(cbase, nslabs, lo, hi):
            ne = nslabs * 16
            pltpu.sync_copy(src_hbm.at[pl.ds(cbase, ne)], src_v.at[pl.ds(0, ne)])
            pltpu.sync_copy(dst_hbm.at[pl.ds(cbase, ne)], dst_v.at[pl.ds(0, ne)])

            def slab(i, off):
                d16 = dst_v[pl.ds(i * 16, 16)]
                s16 = src_v[pl.ds(i * 16, 16)]
                m = (d16 >= lo) & (d16 < hi)
                mi = m.astype(jnp.int32)
                inc = plsc.cumsum(mi)
                pos = off + inc - mi
                plsc.store_scatter(selp, [pos],
                                   s16 | ((d16 - lo) << 16), mask=m)
                return off + jnp.sum(mi)

            off = lax.fori_loop(0, nslabs, slab, 0)
            rpad = jnp.full((16,), R << 16, jnp.int32)
            for j in range(K // 16):
                selp[pl.ds(off + j * 16, 16)] = rpad
            nb = (off + K - 1) // K

            def batch(b, _):
                bb = b * K
                for j in range(K // 16):
                    pk = selp[pl.ds(bb + j * 16, 16)]
                    gidx[0, pl.ds(j * 16, 16)] = pk & 0xFFFF
                    sidx[0, pl.ds(j * 16, 16)] = pk >> 16
                pltpu.async_copy(tsrc_hbm.at[gidx.at[0]], rows, sem1).wait()

                def edge(e, _):
                    va, _ = plsc.unpack(rows[e, pl.ds(FA, 32)],
                                        format=plsc.PackFormat.INTERLEAVED)
                    l16 = sidx[0, pl.ds((e // 16) * 16, 16)]
                    espl = jnp.zeros((16,), jnp.int32) + (e % 16)
                    lspl = l16.at[espl].get(mode="promise_in_bounds")
                    vd = plsc.load_gather(adst_v,
                                          [lspl * 4 + (lanes & (NH - 1))])
                    al = va + vd
                    al = jnp.maximum(al, 0.2 * al)
                    w = jnp.where(lanes < NH, jnp.exp(al), 0.0)
                    msg[e, pl.ds(FA, 16)] = w
                    for h in range(NH):
                        wh = w.at[jnp.full((16,), h, jnp.int32)].get(
                            mode="promise_in_bounds")
                        for jj in range(FH // 32):
                            col = h * FH + jj * 32
                            ev, od = plsc.unpack(
                                rows[e, pl.ds(col, 32)],
                                format=plsc.PackFormat.INTERLEAVED)
                            msg[e, pl.ds(col, 16)] = ev * wh
                            msg[e, pl.ds(col + 16, 16)] = od * wh
                    return 0

                lax.fori_loop(0, K, edge, 0)
                pltpu.sync_copy(msg, acc.at[sidx.at[0]], add=True)
                return 0

            lax.fori_loop(0, nb, batch, 0)

        def pass_body(p, _):
            lo = (2 * p + cid) * R
            hi = lo + R
            pltpu.sync_copy(tdst_hbm.at[pl.ds(lo * 4, R * 4)],
                            adst_v.at[pl.ds(0, R * 4)])
            for j in range((8 * 4) // 16):
                adst_v[pl.ds(R * 4 + j * 16, 16)] = jnp.zeros((16,),
                                                              jnp.float32)
            zoff = 0
            zcps = []
            while zoff < TR:
                zn = min(ZR, TR - zoff)
                zcps.append(pltpu.async_copy(
                    zeros_v.at[pl.ds(0, zn)],
                    acc.at[pl.ds(sid * TR + zoff, zn)], sem1))
                zoff += zn
            for cp in zcps:
                cp.wait()
            plsc.subcore_barrier()

            def chunk(ci, _):
                process_chunk(edge_base + ci * CH, CH // 16, lo, hi)
                return 0
            lax.fori_loop(0, ET // CH, chunk, 0)
            if ET % CH:
                process_chunk(edge_base + (ET // CH) * CH, (ET % CH) // 16,
                              lo, hi)
            plsc.subcore_barrier()
            pltpu.sync_copy(acc.at[pl.ds(sid * TR, TR)],
                            out_hbm.at[pl.ds((2 * p + cid) * R + sid * TR, TR)])
            return 0

        lax.fori_loop(0, PASSES, pass_body, 0)

    return sc_kernel


_sc_layer1 = _make_sc_edge_kernel(F=272, FB=288, NH=4, R=4592, PASSES=6)
_sc_layer2 = _make_sc_edge_kernel(F=144, FB=160, NH=1, R=8528, PASSES=3)


def kernel(x, edge_index, batch, W1, att_src1, att_dst1, b1,
           W2, att_src2, att_dst2, b2, gamma, beta):
    src = edge_index[0]
    dst = edge_index[1]
    b1p = _perm32(b1[None])[0]
    b2p = _perm32(b2[None])[0]
    gammap = _perm32(gamma[None])[0]
    betap = _perm32(beta[None])[0]
    W2p = W2.reshape(8, 16, 2, 128).transpose(0, 2, 1, 3).reshape(256, 128)

    tsrc1, tdst1, self1 = _stage_a(x, W1, att_src1, att_dst1)
    tdst1f = jnp.pad(tdst1, ((0, 6 * 2 * 4592 - NPAD), (0, 0))).reshape(-1)
    acc1 = _sc_layer1(src, dst, tsrc1, tdst1f)[:NPAD]
    tsrc2, tdst2, self2 = _stage_c(acc1, _perm_cols(self1, 256), b1p, W2p,
                                   att_src2, att_dst2)
    tdst2f = jnp.pad(tdst2, ((0, 3 * 2 * 8528 - NPAD), (0, 0))).reshape(-1)
    acc2 = _sc_layer2(src, dst, tsrc2, tdst2f)[:NPAD]
    outp = _stage_e(acc2, _perm_cols(self2, 128), b2p, batch, gammap, betap)
    return _iperm32(outp)


# speedup vs baseline: 1.0045x; 1.0045x over previous
import functools

import jax
import jax.numpy as jnp
from jax import lax
from jax.experimental import pallas as pl
from jax.experimental.pallas import tpu as pltpu
from jax.experimental.pallas import tpu_sc as plsc

N_NODES = 50000
N_EDGES = 800000
N_GRAPHS = 64
BN = 512
NPAD = 50176
NBLK = NPAD // BN

NC = 2
NS = 16
ET = N_EDGES // NS
CH = 1024
K = 64
SEL = CH + K
ZR = 16


def _perm32(x):
    bn, n = x.shape
    return x.reshape(bn, n // 32, 16, 2).transpose(0, 1, 3, 2).reshape(bn, n)


def _iperm32(x):
    bn, n = x.shape
    return x.reshape(bn, n // 32, 2, 16).transpose(0, 1, 3, 2).reshape(bn, n)


def _perm_cols(t, n):
    return jnp.concatenate([_perm32(t[:, :n]), t[:, n:]], axis=1)


def _stage_a_body(x_ref, w1_ref, asr_ref, ads_ref, tsrc_ref, tdst_ref, self_ref):
    h = jnp.dot(x_ref[...], w1_ref[...], preferred_element_type=jnp.float32)
    hr = h.reshape(BN, 4, 64)
    a_src = jnp.sum(hr * asr_ref[...][None, :, :], axis=-1)
    a_dst = jnp.sum(hr * ads_ref[...][None, :, :], axis=-1)
    abits = jnp.stack([a_src.astype(jnp.bfloat16),
                       jnp.zeros((BN, 4), jnp.bfloat16)], axis=-1).reshape(BN, 8)
    tsrc_ref[...] = jnp.concatenate(
        [h.astype(jnp.bfloat16), abits,
         jnp.zeros((BN, 24), jnp.bfloat16)], axis=1)
    tdst_ref[...] = a_dst
    al = a_src + a_dst
    w = jnp.exp(jnp.maximum(al, 0.2 * al))
    wb = jnp.broadcast_to(w[:, :, None], (BN, 4, 64)).reshape(BN, 256)
    self_ref[...] = jnp.concatenate(
        [h * wb, w, jnp.zeros((BN, 12), jnp.float32)], axis=1)


def _stage_a(x, W1, att_src1, att_dst1):
    xp = jnp.pad(x, ((0, NPAD - N_NODES), (0, 0)))
    return pl.pallas_call(
        _stage_a_body,
        grid=(NBLK,),
        in_specs=[
            pl.BlockSpec((BN, 58), lambda i: (i, 0)),
            pl.BlockSpec((58, 256), lambda i: (0, 0)),
            pl.BlockSpec((4, 64), lambda i: (0, 0)),
            pl.BlockSpec((4, 64), lambda i: (0, 0)),
        ],
        out_specs=[
            pl.BlockSpec((BN, 288), lambda i: (i, 0)),
            pl.BlockSpec((BN, 4), lambda i: (i, 0)),
            pl.BlockSpec((BN, 272), lambda i: (i, 0)),
        ],
        out_shape=[
            jax.ShapeDtypeStruct((NPAD, 288), jnp.bfloat16),
            jax.ShapeDtypeStruct((NPAD, 4), jnp.float32),
            jax.ShapeDtypeStruct((NPAD, 272), jnp.float32),
        ],
    )(xp, W1, att_src1, att_dst1)


def _stage_c_body(acc_ref, self_ref, b1_ref, w2_ref, asr_ref, ads_ref,
                  tsrc_ref, tdst_ref, self2_ref):
    tot = acc_ref[...] + self_ref[...]
    num = tot[:, :256]
    den = tot[:, 256:260]
    denb = jnp.broadcast_to(den[:, :, None], (BN, 4, 64)).reshape(BN, 256)
    h1 = jnp.maximum(jnp.where(denb > 0, num / denb, 0.0) + b1_ref[...], 0.0)
    h2 = jnp.dot(h1, w2_ref[...], preferred_element_type=jnp.float32)
    a_src = jnp.sum(h2 * asr_ref[...], axis=-1, keepdims=True)
    a_dst = jnp.sum(h2 * ads_ref[...], axis=-1, keepdims=True)
    abits = jnp.concatenate([a_src.astype(jnp.bfloat16),
                             jnp.zeros((BN, 1), jnp.bfloat16)], axis=1)
    tsrc_ref[...] = jnp.concatenate(
        [h2.astype(jnp.bfloat16), abits,
         jnp.zeros((BN, 30), jnp.bfloat16)], axis=1)
    tdst_ref[...] = jnp.concatenate(
        [a_dst, jnp.zeros((BN, 3), jnp.float32)], axis=1)
    al = a_src + a_dst
    w = jnp.exp(jnp.maximum(al, 0.2 * al))
    self2_ref[...] = jnp.concatenate(
        [h2 * w, w, jnp.zeros((BN, 15), jnp.float32)], axis=1)


def _stage_c(acc1, self1, b1p, W2p, att_src2, att_dst2):
    return pl.pallas_call(
        _stage_c_body,
        grid=(NBLK,),
        in_specs=[
            pl.BlockSpec((BN, 272), lambda i: (i, 0)),
            pl.BlockSpec((BN, 272), lambda i: (i, 0)),
            pl.BlockSpec((1, 256), lambda i: (0, 0)),
            pl.BlockSpec((256, 128), lambda i: (0, 0)),
            pl.BlockSpec((1, 128), lambda i: (0, 0)),
            pl.BlockSpec((1, 128), lambda i: (0, 0)),
        ],
        out_specs=[
            pl.BlockSpec((BN, 160), lambda i: (i, 0)),
            pl.BlockSpec((BN, 4), lambda i: (i, 0)),
            pl.BlockSpec((BN, 144), lambda i: (i, 0)),
        ],
        out_shape=[
            jax.ShapeDtypeStruct((NPAD, 160), jnp.bfloat16),
            jax.ShapeDtypeStruct((NPAD, 4), jnp.float32),
            jax.ShapeDtypeStruct((NPAD, 144), jnp.float32),
        ],
    )(acc1, self1, b1p[None], W2p, att_src2, att_dst2)


def _stage_e_body(acc_ref, self_ref, b2_ref, batch_ref, g_ref, be_ref,
                  out_ref, pooled_ref):
    i = pl.program_id(0)

    @pl.when(i == 0)
    def _():
        pooled_ref[...] = jnp.zeros((N_GRAPHS, 256), jnp.float32)

    tot = acc_ref[...] + self_ref[...]
    den = tot[:, 128:129]
    h2 = jnp.maximum(jnp.where(den > 0, tot[:, :128] / den, 0.0) + b2_ref[...],
                     0.0)
    bvec = batch_ref[0, 0, :]
    gids = lax.broadcasted_iota(jnp.int32, (N_GRAPHS, BN), 0)
    m = (bvec[None, :] == gids).astype(jnp.float32)
    hext = jnp.concatenate(
        [h2, jnp.ones((BN, 1), jnp.float32), jnp.zeros((BN, 127), jnp.float32)],
        axis=1)
    pooled_ref[...] += jnp.dot(m, hext, preferred_element_type=jnp.float32)

    @pl.when(i == NBLK - 1)
    def _():
        acc = pooled_ref[...]
        sums = acc[:, :128]
        counts = acc[:, 128:129]
        pooled = sums / jnp.clip(counts, 1.0, None)
        mu = jnp.mean(pooled, axis=0, keepdims=True)
        var = jnp.mean((pooled - mu) ** 2, axis=0, keepdims=True)
        norm = (pooled - mu) / jnp.sqrt(var + 1e-5)
        out_ref[...] = norm * g_ref[...] + be_ref[...]


def _stage_e(acc2, self2, b2p, batch, gamma, beta):
    bpad = jnp.pad(batch, (0, NPAD - N_NODES), constant_values=N_GRAPHS)
    b3d = bpad.reshape(NBLK, BN)[:, None, :]
    return pl.pallas_call(
        _stage_e_body,
        grid=(NBLK,),
        in_specs=[
            pl.BlockSpec((BN, 144), lambda i: (i, 0)),
            pl.BlockSpec((BN, 144), lambda i: (i, 0)),
            pl.BlockSpec((1, 128), lambda i: (0, 0)),
            pl.BlockSpec((1, 1, BN), lambda i: (i, 0, 0)),
            pl.BlockSpec((1, 128), lambda i: (0, 0)),
            pl.BlockSpec((1, 128), lambda i: (0, 0)),
        ],
        out_specs=pl.BlockSpec((N_GRAPHS, 128), lambda i: (0, 0)),
        out_shape=jax.ShapeDtypeStruct((N_GRAPHS, 128), jnp.float32),
        scratch_shapes=[pltpu.VMEM((N_GRAPHS, 256), jnp.float32)],
    )(acc2, self2, b2p[None], b3d, gamma[None], beta[None])


def _make_sc_edge_kernel(F, FB, NH, R, PASSES):
    FA = F - 16
    FH = FA // NH
    TR = R // NS
    NPR = PASSES * 2 * R

    mesh = plsc.VectorSubcoreMesh(core_axis_name="c", subcore_axis_name="s")

    @functools.partial(
        pl.kernel,
        out_type=jax.ShapeDtypeStruct((NPR, F), jnp.float32),
        mesh=mesh,
        compiler_params=pltpu.CompilerParams(needs_layout_passes=False,
                                             use_tc_tiling_on_sc=False),
        scratch_types=[
            pltpu.VMEM((CH,), jnp.int32),
            pltpu.VMEM((CH,), jnp.int32),
            pltpu.VMEM((SEL,), jnp.int32),
            pltpu.VMEM((1, K), jnp.int32),
            pltpu.VMEM((1, K), jnp.int32),
            pltpu.VMEM((K, FB), jnp.bfloat16),
            pltpu.VMEM(((R + 8) * 4,), jnp.float32),
            pltpu.VMEM((K, F), jnp.float32),
            pltpu.VMEM((ZR, F), jnp.float32),
            pltpu.VMEM_SHARED((R + 8, F), jnp.float32),
            pltpu.SemaphoreType.DMA,
            pltpu.SemaphoreType.DMA,
        ],
    )
    def sc_kernel(src_hbm, dst_hbm, tsrc_hbm, tdst_hbm, out_hbm,
                  src_v, dst_v, selp, gidx, sidx, rows, adst_v, msg,
                  zeros_v, acc, sem1, sem2):
        cid = lax.axis_index("c")
        sid = lax.axis_index("s")
        lanes = lax.iota(jnp.int32, 16)
        zf = jnp.zeros((16,), jnp.float32)

        def zrow(i, _):
            for j in range(F // 16):
                zeros_v[i, pl.ds(j * 16, 16)] = zf
            return 0
        lax.fori_loop(0, ZR, zrow, 0)

        edge_base = sid * ET

        def process_chunk(cbase, nslabs, lo, hi):
            ne = nslabs * 16
            pltpu.sync_copy(src_hbm.at[pl.ds(cbase, ne)], src_v.at[pl.ds(0, ne)])
            pltpu.sync_copy(dst_hbm.at[pl.ds(cbase, ne)], dst_v.at[pl.ds(0, ne)])

            def slab(i, off):
                d16 = dst_v[pl.ds(i * 16, 16)]
                s16 = src_v[pl.ds(i * 16, 16)]
                m = (d16 >= lo) & (d16 < hi)
                mi = m.astype(jnp.int32)
                inc = plsc.cumsum(mi)
                pos = off + inc - mi
                plsc.store_scatter(selp, [pos],
                                   s16 | ((d16 - lo) << 16), mask=m)
                return off + jnp.sum(mi)

            off = lax.fori_loop(0, nslabs, slab, 0)
            rpad = jnp.full((16,), R << 16, jnp.int32)
            for j in range(K // 16):
                selp[pl.ds(off + j * 16, 16)] = rpad
            nb = (off + K - 1) // K

            def batch(b, _):
                bb = b * K
                for j in range(K // 16):
                    pk = selp[pl.ds(bb + j * 16, 16)]
                    gidx[0, pl.ds(j * 16, 16)] = pk & 0xFFFF
                    sidx[0, pl.ds(j * 16, 16)] = pk >> 16
                pltpu.async_copy(tsrc_hbm.at[gidx.at[0]], rows, sem1).wait()

                def slab16(sb, _):
                    l16 = sidx[0, pl.ds(sb * 16, 16)]
                    lbase = l16 * 4
                    for k in range(16):
                        e = sb * 16 + k
                        va, _ = plsc.unpack(rows[e, pl.ds(FA, 32)],
                                            format=plsc.PackFormat.INTERLEAVED)
                        lspl = lbase.at[jnp.full((16,), k, jnp.int32)].get(
                            mode="promise_in_bounds")
                        vd = plsc.load_gather(adst_v,
                                              [lspl + (lanes & (NH - 1))])
                        al = va + vd
                        al = jnp.maximum(al, 0.2 * al)
                        w = jnp.where(lanes < NH, jnp.exp(al), 0.0)
                        msg[e, pl.ds(FA, 16)] = w
                        for h in range(NH):
                            wh = w.at[jnp.full((16,), h, jnp.int32)].get(
                                mode="promise_in_bounds")
                            for jj in range(FH // 32):
                                col = h * FH + jj * 32
                                ev, od = plsc.unpack(
                                    rows[e, pl.ds(col, 32)],
                                    format=plsc.PackFormat.INTERLEAVED)
                                msg[e, pl.ds(col, 16)] = ev * wh
                                msg[e, pl.ds(col + 16, 16)] = od * wh
                    return 0

                lax.fori_loop(0, K // 16, slab16, 0)
                pltpu.sync_copy(msg, acc.at[sidx.at[0]], add=True)
                return 0

            lax.fori_loop(0, nb, batch, 0)

        def pass_body(p, _):
            lo = (2 * p + cid) * R
            hi = lo + R
            pltpu.sync_copy(tdst_hbm.at[pl.ds(lo * 4, R * 4)],
                            adst_v.at[pl.ds(0, R * 4)])
            for j in range((8 * 4) // 16):
                adst_v[pl.ds(R * 4 + j * 16, 16)] = jnp.zeros((16,),
                                                              jnp.float32)
            zoff = 0
            zcps = []
            while zoff < TR:
                zn = min(ZR, TR - zoff)
                zcps.append(pltpu.async_copy(
                    zeros_v.at[pl.ds(0, zn)],
                    acc.at[pl.ds(sid * TR + zoff, zn)], sem1))
                zoff += zn
            for cp in zcps:
                cp.wait()
            plsc.subcore_barrier()

            def chunk(ci, _):
                process_chunk(edge_base + ci * CH, CH // 16, lo, hi)
                return 0
            lax.fori_loop(0, ET // CH, chunk, 0)
            if ET % CH:
                process_chunk(edge_base + (ET // CH) * CH, (ET % CH) // 16,
                              lo, hi)
            plsc.subcore_barrier()
            pltpu.sync_copy(acc.at[pl.ds(sid * TR, TR)],
                            out_hbm.at[pl.ds((2 * p + cid) * R + sid * TR, TR)])
            return 0

        lax.fori_loop(0, PASSES, pass_body, 0)

    return sc_kernel


_sc_layer1 = _make_sc_edge_kernel(F=272, FB=288, NH=4, R=4592, PASSES=6)
_sc_layer2 = _make_sc_edge_kernel(F=144, FB=160, NH=1, R=8528, PASSES=3)


def kernel(x, edge_index, batch, W1, att_src1, att_dst1, b1,
           W2, att_src2, att_dst2, b2, gamma, beta):
    src = edge_index[0]
    dst = edge_index[1]
    b1p = _perm32(b1[None])[0]
    b2p = _perm32(b2[None])[0]
    gammap = _perm32(gamma[None])[0]
    betap = _perm32(beta[None])[0]
    W2p = W2.reshape(8, 16, 2, 128).transpose(0, 2, 1, 3).reshape(256, 128)

    tsrc1, tdst1, self1 = _stage_a(x, W1, att_src1, att_dst1)
    tdst1f = jnp.pad(tdst1, ((0, 6 * 2 * 4592 - NPAD), (0, 0))).reshape(-1)
    acc1 = _sc_layer1(src, dst, tsrc1, tdst1f)[:NPAD]
    tsrc2, tdst2, self2 = _stage_c(acc1, _perm_cols(self1, 256), b1p, W2p,
                                   att_src2, att_dst2)
    tdst2f = jnp.pad(tdst2, ((0, 3 * 2 * 8528 - NPAD), (0, 0))).reshape(-1)
    acc2 = _sc_layer2(src, dst, tsrc2, tdst2f)[:NPAD]
    outp = _stage_e(acc2, _perm_cols(self2, 128), b2p, batch, gammap, betap)
    return _iperm32(outp)


# speedup vs baseline: 1.6102x; 1.6031x over previous
import functools

import jax
import jax.numpy as jnp
from jax import lax
from jax.experimental import pallas as pl
from jax.experimental.pallas import tpu as pltpu
from jax.experimental.pallas import tpu_sc as plsc

N_NODES = 50000
N_EDGES = 800000
N_GRAPHS = 64
BN = 512
NPAD = 50176
NBLK = NPAD // BN

NC = 2
NS = 16
ET = N_EDGES // NS
CH = 1024
K = 64
SEL = CH + K
ZR = 16


def _perm32(x):
    bn, n = x.shape
    return x.reshape(bn, n // 32, 16, 2).transpose(0, 1, 3, 2).reshape(bn, n)


def _iperm32(x):
    bn, n = x.shape
    return x.reshape(bn, n // 32, 2, 16).transpose(0, 1, 3, 2).reshape(bn, n)


def _perm_cols(t, n):
    return jnp.concatenate([_perm32(t[:, :n]), t[:, n:]], axis=1)


def _stage_a_body(x_ref, w1_ref, asr_ref, ads_ref, tsrc_ref, tdst_ref, self_ref):
    h = jnp.dot(x_ref[...], w1_ref[...], preferred_element_type=jnp.float32)
    hr = h.reshape(BN, 4, 64)
    a_src = jnp.sum(hr * asr_ref[...][None, :, :], axis=-1)
    a_dst = jnp.sum(hr * ads_ref[...][None, :, :], axis=-1)
    abits = jnp.stack([a_src.astype(jnp.bfloat16),
                       jnp.zeros((BN, 4), jnp.bfloat16)], axis=-1).reshape(BN, 8)
    tsrc_ref[...] = jnp.concatenate(
        [h.astype(jnp.bfloat16), abits,
         jnp.zeros((BN, 24), jnp.bfloat16)], axis=1)
    z12 = jnp.zeros((BN, 12), jnp.float32)
    tdst_ref[...] = jnp.concatenate([a_dst, z12], axis=1)
    al = a_src + a_dst
    w = jnp.exp(jnp.maximum(al, 0.2 * al))
    wb = jnp.broadcast_to(w[:, :, None], (BN, 4, 64)).reshape(BN, 256)
    self_ref[...] = jnp.concatenate([h * wb, w, z12], axis=1)


def _stage_a(x, W1, att_src1, att_dst1):
    xp = jnp.pad(x, ((0, NPAD - N_NODES), (0, 0)))
    return pl.pallas_call(
        _stage_a_body,
        grid=(NBLK,),
        in_specs=[
            pl.BlockSpec((BN, 58), lambda i: (i, 0)),
            pl.BlockSpec((58, 256), lambda i: (0, 0)),
            pl.BlockSpec((4, 64), lambda i: (0, 0)),
            pl.BlockSpec((4, 64), lambda i: (0, 0)),
        ],
        out_specs=[
            pl.BlockSpec((BN, 288), lambda i: (i, 0)),
            pl.BlockSpec((BN, 16), lambda i: (i, 0)),
            pl.BlockSpec((BN, 272), lambda i: (i, 0)),
        ],
        out_shape=[
            jax.ShapeDtypeStruct((NPAD, 288), jnp.bfloat16),
            jax.ShapeDtypeStruct((NPAD, 16), jnp.float32),
            jax.ShapeDtypeStruct((NPAD, 272), jnp.float32),
        ],
    )(xp, W1, att_src1, att_dst1)


def _stage_c_body(acc_ref, self_ref, b1_ref, w2_ref, asr_ref, ads_ref,
                  tsrc_ref, tdst_ref, self2_ref):
    tot = acc_ref[...] + self_ref[...]
    num = tot[:, :256]
    den = tot[:, 256:260]
    denb = jnp.broadcast_to(den[:, :, None], (BN, 4, 64)).reshape(BN, 256)
    h1 = jnp.maximum(jnp.where(denb > 0, num / denb, 0.0) + b1_ref[...], 0.0)
    h2 = jnp.dot(h1, w2_ref[...], preferred_element_type=jnp.float32)
    a_src = jnp.sum(h2 * asr_ref[...], axis=-1, keepdims=True)
    a_dst = jnp.sum(h2 * ads_ref[...], axis=-1, keepdims=True)
    abits = jnp.concatenate([a_src.astype(jnp.bfloat16),
                             jnp.zeros((BN, 1), jnp.bfloat16)], axis=1)
    tsrc_ref[...] = jnp.concatenate(
        [h2.astype(jnp.bfloat16), abits,
         jnp.zeros((BN, 30), jnp.bfloat16)], axis=1)
    z15 = jnp.zeros((BN, 15), jnp.float32)
    tdst_ref[...] = jnp.concatenate([a_dst, z15], axis=1)
    al = a_src + a_dst
    w = jnp.exp(jnp.maximum(al, 0.2 * al))
    self2_ref[...] = jnp.concatenate([h2 * w, w, z15], axis=1)


def _stage_c(acc1, self1, b1p, W2p, att_src2, att_dst2):
    return pl.pallas_call(
        _stage_c_body,
        grid=(NBLK,),
        in_specs=[
            pl.BlockSpec((BN, 272), lambda i: (i, 0)),
            pl.BlockSpec((BN, 272), lambda i: (i, 0)),
            pl.BlockSpec((1, 256), lambda i: (0, 0)),
            pl.BlockSpec((256, 128), lambda i: (0, 0)),
            pl.BlockSpec((1, 128), lambda i: (0, 0)),
            pl.BlockSpec((1, 128), lambda i: (0, 0)),
        ],
        out_specs=[
            pl.BlockSpec((BN, 160), lambda i: (i, 0)),
            pl.BlockSpec((BN, 16), lambda i: (i, 0)),
            pl.BlockSpec((BN, 144), lambda i: (i, 0)),
        ],
        out_shape=[
            jax.ShapeDtypeStruct((NPAD, 160), jnp.bfloat16),
            jax.ShapeDtypeStruct((NPAD, 16), jnp.float32),
            jax.ShapeDtypeStruct((NPAD, 144), jnp.float32),
        ],
    )(acc1, self1, b1p[None], W2p, att_src2, att_dst2)


def _stage_e_body(acc_ref, self_ref, b2_ref, batch_ref, g_ref, be_ref,
                  out_ref, pooled_ref):
    i = pl.program_id(0)

    @pl.when(i == 0)
    def _():
        pooled_ref[...] = jnp.zeros((N_GRAPHS, 256), jnp.float32)

    tot = acc_ref[...] + self_ref[...]
    den = tot[:, 128:129]
    h2 = jnp.maximum(jnp.where(den > 0, tot[:, :128] / den, 0.0) + b2_ref[...],
                     0.0)
    bvec = batch_ref[0, 0, :]
    gids = lax.broadcasted_iota(jnp.int32, (N_GRAPHS, BN), 0)
    m = (bvec[None, :] == gids).astype(jnp.float32)
    hext = jnp.concatenate(
        [h2, jnp.ones((BN, 1), jnp.float32), jnp.zeros((BN, 127), jnp.float32)],
        axis=1)
    pooled_ref[...] += jnp.dot(m, hext, preferred_element_type=jnp.float32)

    @pl.when(i == NBLK - 1)
    def _():
        acc = pooled_ref[...]
        sums = acc[:, :128]
        counts = acc[:, 128:129]
        pooled = sums / jnp.clip(counts, 1.0, None)
        mu = jnp.mean(pooled, axis=0, keepdims=True)
        var = jnp.mean((pooled - mu) ** 2, axis=0, keepdims=True)
        norm = (pooled - mu) / jnp.sqrt(var + 1e-5)
        out_ref[...] = norm * g_ref[...] + be_ref[...]


def _stage_e(acc2, self2, b2p, batch, gamma, beta):
    bpad = jnp.pad(batch, (0, NPAD - N_NODES), constant_values=N_GRAPHS)
    b3d = bpad.reshape(NBLK, BN)[:, None, :]
    return pl.pallas_call(
        _stage_e_body,
        grid=(NBLK,),
        in_specs=[
            pl.BlockSpec((BN, 144), lambda i: (i, 0)),
            pl.BlockSpec((BN, 144), lambda i: (i, 0)),
            pl.BlockSpec((1, 128), lambda i: (0, 0)),
            pl.BlockSpec((1, 1, BN), lambda i: (i, 0, 0)),
            pl.BlockSpec((1, 128), lambda i: (0, 0)),
            pl.BlockSpec((1, 128), lambda i: (0, 0)),
        ],
        out_specs=pl.BlockSpec((N_GRAPHS, 128), lambda i: (0, 0)),
        out_shape=jax.ShapeDtypeStruct((N_GRAPHS, 128), jnp.float32),
        scratch_shapes=[pltpu.VMEM((N_GRAPHS, 256), jnp.float32)],
    )(acc2, self2, b2p[None], b3d, gamma[None], beta[None])


def _make_sc_edge_kernel(F, FB, NH, R, PASSES):
    FA = F - 16
    FH = FA // NH
    TR = R // NS
    NPR = PASSES * 2 * R

    mesh = plsc.VectorSubcoreMesh(core_axis_name="c", subcore_axis_name="s")

    @functools.partial(
        pl.kernel,
        out_type=jax.ShapeDtypeStruct((NPR, F), jnp.float32),
        mesh=mesh,
        compiler_params=pltpu.CompilerParams(needs_layout_passes=False,
                                             use_tc_tiling_on_sc=False),
        scratch_types=[
            pltpu.VMEM((CH,), jnp.int32),
            pltpu.VMEM((CH,), jnp.int32),
            pltpu.VMEM((SEL,), jnp.int32),
            pltpu.VMEM((1, K), jnp.int32),
            pltpu.VMEM((1, K), jnp.int32),
            pltpu.VMEM((1, K), jnp.int32),
            pltpu.VMEM((K, FB), jnp.bfloat16),
            pltpu.VMEM((K, 16), jnp.float32),
            pltpu.VMEM((K, F), jnp.float32),
            pltpu.VMEM((ZR, F), jnp.float32),
            pltpu.VMEM_SHARED((R + 8, F), jnp.float32),
            pltpu.SemaphoreType.DMA,
            pltpu.SemaphoreType.DMA,
        ],
    )
    def sc_kernel(src_hbm, dst_hbm, tsrc_hbm, tdst_hbm, out_hbm,
                  src_v, dst_v, selp, gidx, didx, sidx, rows, drows, msg,
                  zeros_v, acc, sem1, sem2):
        cid = lax.axis_index("c")
        sid = lax.axis_index("s")
        lanes = lax.iota(jnp.int32, 16)
        zf = jnp.zeros((16,), jnp.float32)

        def zrow(i, _):
            for j in range(F // 16):
                zeros_v[i, pl.ds(j * 16, 16)] = zf
            return 0
        lax.fori_loop(0, ZR, zrow, 0)

        edge_base = sid * ET

        def process_chunk(cbase, nslabs, lo, hi):
            ne = nslabs * 16
            pltpu.sync_copy(src_hbm.at[pl.ds(cbase, ne)], src_v.at[pl.ds(0, ne)])
            pltpu.sync_copy(dst_hbm.at[pl.ds(cbase, ne)], dst_v.at[pl.ds(0, ne)])

            def slab(i, off):
                d16 = dst_v[pl.ds(i * 16, 16)]
                s16 = src_v[pl.ds(i * 16, 16)]
                m = (d16 >= lo) & (d16 < hi)
                mi = m.astype(jnp.int32)
                inc = plsc.cumsum(mi)
                pos = off + inc - mi
                plsc.store_scatter(selp, [pos],
                                   s16 | ((d16 - lo) << 16), mask=m)
                return off + jnp.sum(mi)

            off = lax.fori_loop(0, nslabs, slab, 0)
            rpad = jnp.full((16,), R << 16, jnp.int32)
            for j in range(K // 16):
                selp[pl.ds(off + j * 16, 16)] = rpad
            nb = (off + K - 1) // K

            def batch(b, _):
                bb = b * K
                for j in range(K // 16):
                    pk = selp[pl.ds(bb + j * 16, 16)]
                    g = pk & 0xFFFF
                    l = pk >> 16
                    gidx[0, pl.ds(j * 16, 16)] = g
                    didx[0, pl.ds(j * 16, 16)] = jnp.minimum(l + lo, NPAD - 1)
                    sidx[0, pl.ds(j * 16, 16)] = l
                cp1 = pltpu.async_copy(tsrc_hbm.at[gidx.at[0]], rows, sem1)
                cp2 = pltpu.async_copy(tdst_hbm.at[didx.at[0]], drows, sem2)
                cp1.wait()
                cp2.wait()

                def edge(e, _):
                    va, _ = plsc.unpack(rows[e, pl.ds(FA, 32)],
                                        format=plsc.PackFormat.INTERLEAVED)
                    vd = drows[e, pl.ds(0, 16)]
                    al = va + vd
                    al = jnp.maximum(al, 0.2 * al)
                    w = jnp.where(lanes < NH, jnp.exp(al), 0.0)
                    msg[e, pl.ds(FA, 16)] = w
                    for h in range(NH):
                        wh = w.at[jnp.full((16,), h, jnp.int32)].get(
                            mode="promise_in_bounds")
                        for jj in range(FH // 32):
                            col = h * FH + jj * 32
                            ev, od = plsc.unpack(
                                rows[e, pl.ds(col, 32)],
                                format=plsc.PackFormat.INTERLEAVED)
                            msg[e, pl.ds(col, 16)] = ev * wh
                            msg[e, pl.ds(col + 16, 16)] = od * wh
                    return 0

                lax.fori_loop(0, K, edge, 0)
                pltpu.sync_copy(msg, acc.at[sidx.at[0]], add=True)
                return 0

            lax.fori_loop(0, nb, batch, 0)

        def pass_body(p, _):
            lo = (2 * p + cid) * R
            hi = lo + R
            zoff = 0
            zcps = []
            while zoff < TR:
                zn = min(ZR, TR - zoff)
                zcps.append(pltpu.async_copy(
                    zeros_v.at[pl.ds(0, zn)],
                    acc.at[pl.ds(sid * TR + zoff, zn)], sem1))
                zoff += zn
            for cp in zcps:
                cp.wait()
            plsc.subcore_barrier()

            def chunk(ci, _):
                process_chunk(edge_base + ci * CH, CH // 16, lo, hi)
                return 0
            lax.fori_loop(0, ET // CH, chunk, 0)
            if ET % CH:
                process_chunk(edge_base + (ET // CH) * CH, (ET % CH) // 16,
                              lo, hi)
            plsc.subcore_barrier()
            pltpu.sync_copy(acc.at[pl.ds(sid * TR, TR)],
                            out_hbm.at[pl.ds((2 * p + cid) * R + sid * TR, TR)])
            return 0

        lax.fori_loop(0, PASSES, pass_body, 0)

    return sc_kernel


_sc_layer1 = _make_sc_edge_kernel(F=272, FB=288, NH=4, R=5600, PASSES=5)
_sc_layer2 = _make_sc_edge_kernel(F=144, FB=160, NH=1, R=12208, PASSES=3)


def kernel(x, edge_index, batch, W1, att_src1, att_dst1, b1,
           W2, att_src2, att_dst2, b2, gamma, beta):
    src = edge_index[0]
    dst = edge_index[1]
    b1p = _perm32(b1[None])[0]
    b2p = _perm32(b2[None])[0]
    gammap = _perm32(gamma[None])[0]
    betap = _perm32(beta[None])[0]
    W2p = W2.reshape(8, 16, 2, 128).transpose(0, 2, 1, 3).reshape(256, 128)

    tsrc1, tdst1, self1 = _stage_a(x, W1, att_src1, att_dst1)
    acc1 = _sc_layer1(src, dst, tsrc1, tdst1)[:NPAD]
    tsrc2, tdst2, self2 = _stage_c(acc1, _perm_cols(self1, 256), b1p, W2p,
                                   att_src2, att_dst2)
    acc2 = _sc_layer2(src, dst, tsrc2, tdst2)[:NPAD]
    outp = _stage_e(acc2, _perm_cols(self2, 128), b2p, batch, gammap, betap)
    return _iperm32(outp)
